# Initial kernel scaffold; baseline (speedup 1.0000x reference)
import jax
import jax.numpy as jnp
from jax.experimental import pallas as pl


def kernel(x, idx_mask):
    raise NotImplementedError("write your pallas kernel here")


# speedup vs baseline: 1.3201x; 1.3201x over previous
import jax
import jax.numpy as jnp
from jax import lax
from jax.experimental import pallas as pl
from jax.experimental.pallas import tpu as pltpu
from jax.experimental.pallas import tpu_sc as plsc


_BLK = 2000


def _copy_mean_body(x_ref, o_ref, tok_ref, acc_ref, *, nblk, n_rows):
    i = pl.program_id(0)
    blk = x_ref[...]
    o_ref[...] = blk

    @pl.when(i == 0)
    def _init():
        acc_ref[...] = jnp.zeros_like(acc_ref)

    acc_ref[...] += jnp.sum(
        blk.reshape(blk.shape[0] // 8, 8, blk.shape[1]), axis=0
    )

    @pl.when(i == nblk - 1)
    def _fin():
        tok_ref[...] = jnp.sum(acc_ref[...], axis=0, keepdims=True) * (
            1.0 / n_rows
        )


def _copy_and_mean(x):
    n, d = x.shape
    blk = _BLK if n % _BLK == 0 else 8
    nblk = n // blk
    import functools

    body = functools.partial(_copy_mean_body, nblk=nblk, n_rows=n)
    return pl.pallas_call(
        body,
        grid=(nblk,),
        in_specs=[pl.BlockSpec((blk, d), lambda i: (i, 0))],
        out_specs=[
            pl.BlockSpec((blk, d), lambda i: (i, 0)),
            pl.BlockSpec((1, d), lambda i: (0, 0)),
        ],
        out_shape=[
            jax.ShapeDtypeStruct((n, d), x.dtype),
            jax.ShapeDtypeStruct((1, d), jnp.float32),
        ],
        scratch_shapes=[pltpu.VMEM((8, d), jnp.float32)],
        compiler_params=pltpu.CompilerParams(
            dimension_semantics=("arbitrary",)
        ),
    )(x)


_NC, _NS = 2, 16
_NW = _NC * _NS


def _make_scatter(n, d, m_pad):
    per = m_pad // _NW
    nch = 1
    while per // nch > 128 or per % nch or (per // nch) % 8:
        nch += 1
    ch = per // nch
    mesh = plsc.VectorSubcoreMesh(core_axis_name="c", subcore_axis_name="s")

    def body(out_hbm, tok_hbm, idx_hbm, rows_v, tok_v, idx_vs, sem_i, sem_s):
        wid = lax.axis_index("s") * _NC + lax.axis_index("c")
        base = wid * per
        loads = [
            pltpu.async_copy(
                idx_hbm.at[pl.ds(base + j * ch, ch)], idx_vs[j], sem_i
            )
            for j in range(nch)
        ]
        pltpu.sync_copy(tok_hbm, tok_v)
        tv = [tok_v[pl.ds(16 * c, 16)] for c in range(d // 16)]

        @pl.loop(0, ch)
        def _fill(r):
            for c in range(d // 16):
                rows_v[r, pl.ds(16 * c, 16)] = tv[c]

        for l in loads:
            l.wait()
        stores = [
            pltpu.async_copy(rows_v, out_hbm.at[idx_vs[j]], sem_s)
            for j in range(nch)
        ]
        for st in stores:
            st.wait()

    return pl.kernel(
        body,
        out_type=(),
        mesh=mesh,
        scratch_types=[
            pltpu.VMEM((ch, d), jnp.float32),
            pltpu.VMEM((d,), jnp.float32),
            [pltpu.VMEM((ch,), jnp.int32) for _ in range(nch)],
            pltpu.SemaphoreType.DMA,
            pltpu.SemaphoreType.DMA,
        ],
    )


def kernel(x, idx_mask):
    n, d = x.shape
    m = idx_mask.shape[0]
    m_pad = -(-m // (8 * _NW)) * (8 * _NW)
    if m_pad != m:
        idx_mask = jnp.concatenate(
            [idx_mask, jnp.broadcast_to(idx_mask[:1], (m_pad - m,))]
        )
    out, token = _copy_and_mean(x)
    out_ref = jax.new_ref(out)
    _make_scatter(n, d, m_pad)(out_ref, token.reshape(d), idx_mask)
    return jax.freeze(out_ref)


# speedup vs baseline: 1.7356x; 1.3148x over previous
import jax
import jax.numpy as jnp
from jax import lax
from jax.experimental import pallas as pl
from jax.experimental.pallas import tpu as pltpu
from jax.experimental.pallas import tpu_sc as plsc


_BLK = 10000


def _copy_mean_body(x_ref, o_ref, tok_ref, acc_ref, *, nblk, n_rows):
    i = pl.program_id(0)
    blk = x_ref[...]
    o_ref[...] = blk

    @pl.when(i == 0)
    def _init():
        acc_ref[...] = jnp.zeros_like(acc_ref)

    acc_ref[...] += jnp.sum(
        blk.reshape(blk.shape[0] // 8, 8, blk.shape[1]), axis=0
    )

    @pl.when(i == nblk - 1)
    def _fin():
        tok_ref[...] = jnp.sum(acc_ref[...], axis=0, keepdims=True) * (
            1.0 / n_rows
        )


def _copy_and_mean(x):
    n, d = x.shape
    blk = _BLK if n % _BLK == 0 else 8
    nblk = n // blk
    import functools

    body = functools.partial(_copy_mean_body, nblk=nblk, n_rows=n)
    return pl.pallas_call(
        body,
        grid=(nblk,),
        in_specs=[pl.BlockSpec((blk, d), lambda i: (i, 0))],
        out_specs=[
            pl.BlockSpec((blk, d), lambda i: (i, 0)),
            pl.BlockSpec((1, d), lambda i: (0, 0)),
        ],
        out_shape=[
            jax.ShapeDtypeStruct((n, d), x.dtype),
            jax.ShapeDtypeStruct((1, d), jnp.float32),
        ],
        scratch_shapes=[pltpu.VMEM((8, d), jnp.float32)],
        compiler_params=pltpu.CompilerParams(
            dimension_semantics=("arbitrary",)
        ),
    )(x)


_NC, _NS = 2, 16
_NW = _NC * _NS


def _make_scatter(n, d, m_pad):
    per = m_pad // _NW
    nch = 1
    while per // nch > 128 or per % nch or (per // nch) % 8:
        nch += 1
    ch = per // nch
    mesh = plsc.VectorSubcoreMesh(core_axis_name="c", subcore_axis_name="s")

    def body(out_hbm, tok_hbm, idx_hbm, rows_v, tok_v, idx_vs, sem_i, sem_s):
        wid = lax.axis_index("s") * _NC + lax.axis_index("c")
        base = wid * per
        loads = [
            pltpu.async_copy(
                idx_hbm.at[pl.ds(base + j * ch, ch)], idx_vs[j], sem_i
            )
            for j in range(nch)
        ]
        pltpu.sync_copy(tok_hbm, tok_v)
        tv = [tok_v[pl.ds(16 * c, 16)] for c in range(d // 16)]

        @pl.loop(0, ch)
        def _fill(r):
            for c in range(d // 16):
                rows_v[r, pl.ds(16 * c, 16)] = tv[c]

        for l in loads:
            l.wait()
        stores = [
            pltpu.async_copy(rows_v, out_hbm.at[idx_vs[j]], sem_s)
            for j in range(nch)
        ]
        for st in stores:
            st.wait()

    return pl.kernel(
        body,
        out_type=(),
        mesh=mesh,
        scratch_types=[
            pltpu.VMEM((ch, d), jnp.float32),
            pltpu.VMEM((d,), jnp.float32),
            [pltpu.VMEM((ch,), jnp.int32) for _ in range(nch)],
            pltpu.SemaphoreType.DMA,
            pltpu.SemaphoreType.DMA,
        ],
    )


def kernel(x, idx_mask):
    n, d = x.shape
    m = idx_mask.shape[0]
    m_pad = -(-m // (8 * _NW)) * (8 * _NW)
    if m_pad != m:
        idx_mask = jnp.concatenate(
            [idx_mask, jnp.broadcast_to(idx_mask[:1], (m_pad - m,))]
        )
    out, token = _copy_and_mean(x)
    out_ref = jax.new_ref(out)
    _make_scatter(n, d, m_pad)(out_ref, token.reshape(d), idx_mask)
    return jax.freeze(out_ref)
